# baseline (speedup 1.0000x reference)
import functools

import jax
import jax.numpy as jnp
from jax import lax
from jax.experimental import pallas as pl
from jax.experimental.pallas import tpu as pltpu
from jax.experimental.pallas import tpu_sc as plsc

N_NODES = 10000
N_EDGES = 320000
N_PRED = 320000

NC = 2
NS = 16
NW = NC * NS
CHUNK = 128
EW = N_EDGES // NW
KCH = -(-EW // CHUNK)
EWP = KCH * CHUNK
EP = NW * EWP
NPAD = 10112
RPS = NPAD // NS


NCH = EP // CHUNK


def _pad_idx(a, pad_val):
    a = a.astype(jnp.int32)
    pad = jnp.full((EP - a.shape[0],), pad_val, dtype=jnp.int32)
    return jnp.concatenate([a, pad]).reshape(NCH, CHUNK)


def _sc_aggregate(table, src_idx, dst_idx, width, with_deg):
    assert KCH % 2 == 1 and KCH >= 3
    zeros_w = jnp.zeros((NPAD, width), dtype=jnp.float32)
    out_type = [jax.ShapeDtypeStruct((NC, NPAD, width), jnp.float32)]
    scratch = [
        pltpu.VMEM((KCH, CHUNK), jnp.int32),
        pltpu.VMEM((KCH, CHUNK), jnp.int32),
        pltpu.VMEM((CHUNK, width), jnp.float32),
        pltpu.VMEM((CHUNK, width), jnp.float32),
        pltpu.VMEM_SHARED((NPAD, width), jnp.float32),
        pltpu.SemaphoreType.DMA,
        pltpu.SemaphoreType.DMA,
    ]
    extra_in = ()
    if with_deg:
        zeros_d = jnp.zeros((NPAD, 16), dtype=jnp.float32)
        ones = jnp.ones((CHUNK, 16), dtype=jnp.float32)
        extra_in = (zeros_d, ones)
        out_type.append(jax.ShapeDtypeStruct((NC, NPAD, 16), jnp.float32))
        scratch.extend([
            pltpu.VMEM((CHUNK, 16), jnp.float32),
            pltpu.VMEM_SHARED((NPAD, 16), jnp.float32),
        ])

    @functools.partial(
        pl.kernel,
        out_type=tuple(out_type),
        mesh=plsc.VectorSubcoreMesh(core_axis_name="c", subcore_axis_name="s"),
        scratch_types=scratch,
        compiler_params=pltpu.CompilerParams(use_tc_tiling_on_sc=False),
    )
    def agg_kernel(table_hbm, src_hbm, dst_hbm, zeros_hbm, *rest):
        if with_deg:
            (zerosd_hbm, ones_hbm, out_hbm, outd_hbm,
             src_v, dst_v, rows0, rows1, acc_s, sem0, sem1,
             ones_v, accd_s) = rest
        else:
            (out_hbm, src_v, dst_v, rows0, rows1, acc_s,
             sem0, sem1) = rest
        c = lax.axis_index("c")
        s = lax.axis_index("s")
        wid = c * NS + s
        pltpu.sync_copy(zeros_hbm.at[pl.ds(s * RPS, RPS)],
                        acc_s.at[pl.ds(s * RPS, RPS)])
        pltpu.sync_copy(src_hbm.at[pl.ds(wid * KCH, KCH)], src_v)
        pltpu.sync_copy(dst_hbm.at[pl.ds(wid * KCH, KCH)], dst_v)
        if with_deg:
            pltpu.sync_copy(zerosd_hbm.at[pl.ds(s * RPS, RPS)],
                            accd_s.at[pl.ds(s * RPS, RPS)])
            pltpu.sync_copy(ones_hbm, ones_v)
        plsc.subcore_barrier()

        def scat(rows_v, j):
            pltpu.sync_copy(rows_v, acc_s.at[dst_v.at[j]], add=True)
            if with_deg:
                pltpu.sync_copy(ones_v, accd_s.at[dst_v.at[j]], add=True)

        pltpu.async_copy(table_hbm.at[src_v.at[0]], rows0, sem0)

        def pair(g, carry):
            j0 = 2 * g
            pltpu.async_copy(table_hbm.at[src_v.at[j0 + 1]], rows1, sem1)
            pltpu.make_async_copy(table_hbm.at[src_v.at[j0]],
                                  rows0, sem0).wait()
            scat(rows0, j0)
            pltpu.async_copy(table_hbm.at[src_v.at[j0 + 2]], rows0, sem0)
            pltpu.make_async_copy(table_hbm.at[src_v.at[j0 + 1]],
                                  rows1, sem1).wait()
            scat(rows1, j0 + 1)
            return carry

        lax.fori_loop(0, KCH // 2, pair, 0)
        pltpu.make_async_copy(table_hbm.at[src_v.at[KCH - 1]],
                              rows0, sem0).wait()
        scat(rows0, KCH - 1)
        plsc.subcore_barrier()
        pltpu.sync_copy(acc_s.at[pl.ds(s * RPS, RPS)],
                        out_hbm.at[c, pl.ds(s * RPS, RPS)])
        if with_deg:
            pltpu.sync_copy(accd_s.at[pl.ds(s * RPS, RPS)],
                            outd_hbm.at[c, pl.ds(s * RPS, RPS)])

    return agg_kernel(table, src_idx, dst_idx, zeros_w, *extra_in)


def _sc_decoder_gather(za, zb, s_idx, d_idx, k0, n):
    assert n >= 4
    rows = n * (CHUNK // 4)

    @functools.partial(
        pl.kernel,
        out_type=jax.ShapeDtypeStruct((NW * rows, 128), jnp.float32),
        mesh=plsc.VectorSubcoreMesh(core_axis_name="c", subcore_axis_name="s"),
        scratch_types=[
            pltpu.VMEM((n, CHUNK), jnp.int32),
            pltpu.VMEM((n, CHUNK), jnp.int32),
            pltpu.VMEM((CHUNK, 32), jnp.float32),
            pltpu.VMEM((CHUNK, 32), jnp.float32),
            pltpu.VMEM((CHUNK, 32), jnp.float32),
            pltpu.VMEM((CHUNK, 32), jnp.float32),
            pltpu.VMEM((32, 128), jnp.float32),
            pltpu.VMEM((32, 128), jnp.float32),
            pltpu.SemaphoreType.DMA,
            pltpu.SemaphoreType.DMA,
            pltpu.SemaphoreType.DMA,
            pltpu.SemaphoreType.DMA,
            pltpu.SemaphoreType.DMA,
            pltpu.SemaphoreType.DMA,
        ],
        compiler_params=pltpu.CompilerParams(use_tc_tiling_on_sc=False),
    )
    def dec_kernel(za_hbm, zb_hbm, s_hbm, d_hbm, out_hbm,
                   s_v, d_v, a0, b0, a1, b1, o0, o1,
                   ga0, gb0, ga1, gb1, sem_s0, sem_s1):
        c = lax.axis_index("c")
        s = lax.axis_index("s")
        wid = c * NS + s
        pltpu.sync_copy(s_hbm.at[pl.ds(k0 + wid * n, n)], s_v)
        pltpu.sync_copy(d_hbm.at[pl.ds(k0 + wid * n, n)], d_v)
        base4 = wid * rows

        def gath(j, a_v, b_v, sa, sb):
            pltpu.async_copy(za_hbm.at[s_v.at[j]], a_v, sa)
            pltpu.async_copy(zb_hbm.at[d_v.at[j]], b_v, sb)

        def gwait(j, a_v, b_v, sa, sb):
            pltpu.make_async_copy(za_hbm.at[s_v.at[j]], a_v, sa).wait()
            pltpu.make_async_copy(zb_hbm.at[d_v.at[j]], b_v, sb).wait()

        def relu_into(o_v, a_v, b_v):
            def rbody(q, carry2):
                for i in range(4):
                    for half in range(2):
                        si = pl.ds(16 * half, 16)
                        so = pl.ds(32 * i + 16 * half, 16)
                        o_v[q, so] = jnp.maximum(
                            a_v[4 * q + i, si] + b_v[4 * q + i, si], 0.0)
                return carry2

            lax.fori_loop(0, CHUNK // 4, rbody, 0)

        def store(j, o_v, sem):
            pltpu.async_copy(
                o_v, out_hbm.at[pl.ds(base4 + j * (CHUNK // 4), CHUNK // 4)],
                sem)

        def swait(j, o_v, sem):
            pltpu.make_async_copy(
                o_v, out_hbm.at[pl.ds(base4 + j * (CHUNK // 4), CHUNK // 4)],
                sem).wait()

        gath(0, a0, b0, ga0, gb0)
        gath(1, a1, b1, ga1, gb1)
        gwait(0, a0, b0, ga0, gb0)
        relu_into(o0, a0, b0)
        store(0, o0, sem_s0)
        gath(2, a0, b0, ga0, gb0)
        gwait(1, a1, b1, ga1, gb1)
        relu_into(o1, a1, b1)
        store(1, o1, sem_s1)
        gath(3, a1, b1, ga1, gb1)

        def pair(g, carry):
            j0 = 2 * g
            gwait(j0, a0, b0, ga0, gb0)
            swait(j0 - 2, o0, sem_s0)
            relu_into(o0, a0, b0)
            store(j0, o0, sem_s0)
            gath(j0 + 2, a0, b0, ga0, gb0)
            gwait(j0 + 1, a1, b1, ga1, gb1)
            swait(j0 - 1, o1, sem_s1)
            relu_into(o1, a1, b1)
            store(j0 + 1, o1, sem_s1)
            gath(j0 + 3, a1, b1, ga1, gb1)
            return carry

        if n % 2 == 1:
            lax.fori_loop(1, (n - 3) // 2, pair, 0)
            jm = n - 3
            gwait(jm, a0, b0, ga0, gb0)
            swait(jm - 2, o0, sem_s0)
            relu_into(o0, a0, b0)
            store(jm, o0, sem_s0)
            gath(jm + 2, a0, b0, ga0, gb0)
            gwait(jm + 1, a1, b1, ga1, gb1)
            swait(jm - 1, o1, sem_s1)
            relu_into(o1, a1, b1)
            store(jm + 1, o1, sem_s1)
            gwait(jm + 2, a0, b0, ga0, gb0)
            swait(jm, o0, sem_s0)
            relu_into(o0, a0, b0)
            store(jm + 2, o0, sem_s0)
            swait(jm + 1, o1, sem_s1)
            swait(jm + 2, o0, sem_s0)
        else:
            lax.fori_loop(1, (n - 2) // 2, pair, 0)
            jm = n - 2
            gwait(jm, a0, b0, ga0, gb0)
            swait(jm - 2, o0, sem_s0)
            relu_into(o0, a0, b0)
            store(jm, o0, sem_s0)
            gwait(jm + 1, a1, b1, ga1, gb1)
            swait(jm - 1, o1, sem_s1)
            relu_into(o1, a1, b1)
            store(jm + 1, o1, sem_s1)
            swait(jm, o0, sem_s0)
            swait(jm + 1, o1, sem_s1)

    return dec_kernel(za, zb, s_idx, d_idx)


_RB = 1000
_NBLK = N_NODES // _RB


def _tc_premul(x, W1l, W1r):
    def body(x_ref, wl_ref, wr_ref, p_ref, r_ref):
        xb = x_ref[...]
        p_ref[...] = jnp.dot(xb, wl_ref[...], preferred_element_type=jnp.float32)
        r_ref[...] = jnp.dot(xb, wr_ref[...], preferred_element_type=jnp.float32)

    return pl.pallas_call(
        body,
        grid=(_NBLK,),
        in_specs=[
            pl.BlockSpec((_RB, 128), lambda i: (i, 0)),
            pl.BlockSpec((128, 64), lambda i: (0, 0)),
            pl.BlockSpec((128, 64), lambda i: (0, 0)),
        ],
        out_specs=[
            pl.BlockSpec((_RB, 64), lambda i: (i, 0)),
            pl.BlockSpec((_RB, 64), lambda i: (i, 0)),
        ],
        out_shape=[
            jax.ShapeDtypeStruct((N_NODES, 64), jnp.float32),
            jax.ShapeDtypeStruct((N_NODES, 64), jnp.float32),
        ],
    )(x, W1l, W1r)


def _tc_layer2_premul(part64, part_deg, r1, b1, W2l, W2r):
    def body(p_ref, d_ref, r1_ref, b1_ref, wl_ref, wr_ref, p2_ref, r2_ref):
        agg = p_ref[0] + p_ref[1]
        deg = d_ref[0, :, 0:1] + d_ref[1, :, 0:1]
        invd = 1.0 / jnp.maximum(deg, 1.0)
        h = jnp.maximum(agg * invd + b1_ref[...] + r1_ref[...], 0.0)
        p2_ref[...] = jnp.dot(h, wl_ref[...], preferred_element_type=jnp.float32)
        r2_ref[...] = jnp.dot(h, wr_ref[...], preferred_element_type=jnp.float32)

    return pl.pallas_call(
        body,
        grid=(_NBLK,),
        in_specs=[
            pl.BlockSpec((NC, _RB, 64), lambda i: (0, i, 0)),
            pl.BlockSpec((NC, _RB, 16), lambda i: (0, i, 0)),
            pl.BlockSpec((_RB, 64), lambda i: (i, 0)),
            pl.BlockSpec((1, 64), lambda i: (0, 0)),
            pl.BlockSpec((64, 32), lambda i: (0, 0)),
            pl.BlockSpec((64, 32), lambda i: (0, 0)),
        ],
        out_specs=[
            pl.BlockSpec((_RB, 32), lambda i: (i, 0)),
            pl.BlockSpec((_RB, 32), lambda i: (i, 0)),
        ],
        out_shape=[
            jax.ShapeDtypeStruct((N_NODES, 32), jnp.float32),
            jax.ShapeDtypeStruct((N_NODES, 32), jnp.float32),
        ],
    )(part64, part_deg, r1, b1, W2l, W2r)


def _tc_decoder_tables(part32, part_deg, r2, b2, Wa_s, Wa_d, ba):
    def body(p_ref, d_ref, r2_ref, b2_ref, ws_ref, wd_ref, ba_ref,
             za_ref, zb_ref):
        agg = p_ref[0] + p_ref[1]
        deg = d_ref[0, :, 0:1] + d_ref[1, :, 0:1]
        invd = 1.0 / jnp.maximum(deg, 1.0)
        z = agg * invd + b2_ref[...] + r2_ref[...]
        za_ref[...] = (jnp.dot(z, ws_ref[...], preferred_element_type=jnp.float32)
                       + ba_ref[...])
        zb_ref[...] = jnp.dot(z, wd_ref[...], preferred_element_type=jnp.float32)

    return pl.pallas_call(
        body,
        grid=(_NBLK,),
        in_specs=[
            pl.BlockSpec((NC, _RB, 32), lambda i: (0, i, 0)),
            pl.BlockSpec((NC, _RB, 16), lambda i: (0, i, 0)),
            pl.BlockSpec((_RB, 32), lambda i: (i, 0)),
            pl.BlockSpec((1, 32), lambda i: (0, 0)),
            pl.BlockSpec((32, 32), lambda i: (0, 0)),
            pl.BlockSpec((32, 32), lambda i: (0, 0)),
            pl.BlockSpec((1, 32), lambda i: (0, 0)),
        ],
        out_specs=[
            pl.BlockSpec((_RB, 32), lambda i: (i, 0)),
            pl.BlockSpec((_RB, 32), lambda i: (i, 0)),
        ],
        out_shape=[
            jax.ShapeDtypeStruct((N_NODES, 32), jnp.float32),
            jax.ShapeDtypeStruct((N_NODES, 32), jnp.float32),
        ],
    )(part32, part_deg, r2, b2, Wa_s, Wa_d, ba)


_RB4 = 1024


def _tc_mlp(h1p, Wb4, bb4, Wc4, bc4):
    def body(h_ref, wb_ref, bb_ref, wc_ref, bc_ref, o_ref):
        h2 = jnp.maximum(
            jnp.dot(h_ref[...], wb_ref[...], preferred_element_type=jnp.float32)
            + bb_ref[...], 0.0)
        o_ref[...] = (jnp.dot(h2, wc_ref[...],
                              preferred_element_type=jnp.float32)
                      + bc_ref[...])

    nrows = h1p.shape[0]
    assert nrows % _RB4 == 0
    return pl.pallas_call(
        body,
        grid=(nrows // _RB4,),
        in_specs=[
            pl.BlockSpec((_RB4, 128), lambda i: (i, 0)),
            pl.BlockSpec((128, 64), lambda i: (0, 0)),
            pl.BlockSpec((1, 64), lambda i: (0, 0)),
            pl.BlockSpec((64, 4), lambda i: (0, 0)),
            pl.BlockSpec((1, 4), lambda i: (0, 0)),
        ],
        out_specs=pl.BlockSpec((_RB4, 4), lambda i: (i, 0)),
        out_shape=jax.ShapeDtypeStruct((nrows, 4), jnp.float32),
    )(h1p, Wb4, bb4, Wc4, bc4)


def kernel(x, edge_index, edge_label_index, W1l, b1, W1r, W2l, b2, W2r,
           Wa, ba, Wb, bb, Wc, bc):
    src = _pad_idx(edge_index[0], 0)
    dst = _pad_idx(edge_index[1], N_NODES)
    ls = _pad_idx(edge_label_index[0], 0)
    ld = _pad_idx(edge_label_index[1], 0)

    b1r = b1.reshape(1, 64)
    b2r = b2.reshape(1, 32)
    bar = ba.reshape(1, 32)
    z128 = jnp.zeros((32, 16), jnp.float32)
    z64 = jnp.zeros((16, 1), jnp.float32)
    Wb4 = jnp.block([[Wb if i == j else z128 for j in range(4)]
                     for i in range(4)])
    Wc4 = jnp.block([[Wc if i == j else z64 for j in range(4)]
                     for i in range(4)])
    bb4 = jnp.tile(bb, 4).reshape(1, 64)
    bc4 = jnp.tile(bc, 4).reshape(1, 4)

    p1, r1 = _tc_premul(x, W1l, W1r)
    part64, part_deg = _sc_aggregate(p1, src, dst, 64, with_deg=True)
    p2, r2 = _tc_layer2_premul(part64, part_deg, r1, b1r, W2l, W2r)
    (part32,) = _sc_aggregate(p2, src, dst, 32, with_deg=False)
    za, zb = _tc_decoder_tables(part32, part_deg, r2, b2r,
                                Wa[:32], Wa[32:], bar)
    outs = []
    k0 = 0
    for n in (27, 26, 26):
        h1s = _sc_decoder_gather(za, zb, ls, ld, k0, n)
        outs.append(_tc_mlp(h1s, Wb4, bb4, Wc4, bc4))
        k0 += NW * n
    o = jnp.concatenate(outs, axis=0)
    return o[:N_PRED // 4].reshape(N_PRED)


# speedup vs baseline: 1.1675x; 1.1675x over previous
import functools

import jax
import jax.numpy as jnp
from jax import lax
from jax.experimental import pallas as pl
from jax.experimental.pallas import tpu as pltpu
from jax.experimental.pallas import tpu_sc as plsc

N_NODES = 10000
N_EDGES = 320000
N_PRED = 320000

NC = 2
NS = 16
NW = NC * NS
CHUNK = 128
EW = N_EDGES // NW
KCH = -(-EW // CHUNK)
EWP = KCH * CHUNK
EP = NW * EWP
NPAD = 10112
RPS = NPAD // NS


NCH = EP // CHUNK


def _pad_idx(a, pad_val, spread=False):
    a = a.astype(jnp.int32)
    npad = EP - a.shape[0]
    if spread:
        pad = pad_val + jnp.arange(npad, dtype=jnp.int32) % (NPAD - N_NODES)
    else:
        pad = jnp.full((npad,), pad_val, dtype=jnp.int32)
    return jnp.concatenate([a, pad]).reshape(NCH, CHUNK)


def _sc_aggregate(table, src_idx, dst_idx, width, with_deg):
    assert KCH % 2 == 1 and KCH >= 3
    zeros_w = jnp.zeros((NPAD, width), dtype=jnp.float32)
    out_type = [jax.ShapeDtypeStruct((NC, NPAD, width), jnp.float32)]
    scratch = [
        pltpu.VMEM((KCH, CHUNK), jnp.int32),
        pltpu.VMEM((KCH, CHUNK), jnp.int32),
        pltpu.VMEM((CHUNK, width), jnp.float32),
        pltpu.VMEM((CHUNK, width), jnp.float32),
        pltpu.VMEM_SHARED((NPAD, width), jnp.float32),
        pltpu.SemaphoreType.DMA,
        pltpu.SemaphoreType.DMA,
    ]
    extra_in = ()
    if with_deg:
        zeros_d = jnp.zeros((NPAD, 16), dtype=jnp.float32)
        ones = jnp.ones((CHUNK, 16), dtype=jnp.float32)
        extra_in = (zeros_d, ones)
        out_type.append(jax.ShapeDtypeStruct((NC, NPAD, 16), jnp.float32))
        scratch.extend([
            pltpu.VMEM((CHUNK, 16), jnp.float32),
            pltpu.VMEM_SHARED((NPAD, 16), jnp.float32),
        ])

    @functools.partial(
        pl.kernel,
        out_type=tuple(out_type),
        mesh=plsc.VectorSubcoreMesh(core_axis_name="c", subcore_axis_name="s"),
        scratch_types=scratch,
        compiler_params=pltpu.CompilerParams(use_tc_tiling_on_sc=False),
    )
    def agg_kernel(table_hbm, src_hbm, dst_hbm, zeros_hbm, *rest):
        if with_deg:
            (zerosd_hbm, ones_hbm, out_hbm, outd_hbm,
             src_v, dst_v, rows0, rows1, acc_s, sem0, sem1,
             ones_v, accd_s) = rest
        else:
            (out_hbm, src_v, dst_v, rows0, rows1, acc_s,
             sem0, sem1) = rest
        c = lax.axis_index("c")
        s = lax.axis_index("s")
        wid = c * NS + s
        pltpu.sync_copy(zeros_hbm.at[pl.ds(s * RPS, RPS)],
                        acc_s.at[pl.ds(s * RPS, RPS)])
        pltpu.sync_copy(src_hbm.at[pl.ds(wid * KCH, KCH)], src_v)
        pltpu.sync_copy(dst_hbm.at[pl.ds(wid * KCH, KCH)], dst_v)
        if with_deg:
            pltpu.sync_copy(zerosd_hbm.at[pl.ds(s * RPS, RPS)],
                            accd_s.at[pl.ds(s * RPS, RPS)])
            pltpu.sync_copy(ones_hbm, ones_v)
        plsc.subcore_barrier()

        def scat(rows_v, j):
            pltpu.sync_copy(rows_v, acc_s.at[dst_v.at[j]], add=True)
            if with_deg:
                pltpu.sync_copy(ones_v, accd_s.at[dst_v.at[j]], add=True)

        pltpu.async_copy(table_hbm.at[src_v.at[0]], rows0, sem0)

        def pair(g, carry):
            j0 = 2 * g
            pltpu.async_copy(table_hbm.at[src_v.at[j0 + 1]], rows1, sem1)
            pltpu.make_async_copy(table_hbm.at[src_v.at[j0]],
                                  rows0, sem0).wait()
            scat(rows0, j0)
            pltpu.async_copy(table_hbm.at[src_v.at[j0 + 2]], rows0, sem0)
            pltpu.make_async_copy(table_hbm.at[src_v.at[j0 + 1]],
                                  rows1, sem1).wait()
            scat(rows1, j0 + 1)
            return carry

        lax.fori_loop(0, KCH // 2, pair, 0)
        pltpu.make_async_copy(table_hbm.at[src_v.at[KCH - 1]],
                              rows0, sem0).wait()
        scat(rows0, KCH - 1)
        plsc.subcore_barrier()
        pltpu.sync_copy(acc_s.at[pl.ds(s * RPS, RPS)],
                        out_hbm.at[c, pl.ds(s * RPS, RPS)])
        if with_deg:
            pltpu.sync_copy(accd_s.at[pl.ds(s * RPS, RPS)],
                            outd_hbm.at[c, pl.ds(s * RPS, RPS)])

    return agg_kernel(table, src_idx, dst_idx, zeros_w, *extra_in)


def _sc_decoder_gather(za, zb, s_idx, d_idx, k0, n):
    assert n >= 4
    rows = n * (CHUNK // 4)

    @functools.partial(
        pl.kernel,
        out_type=jax.ShapeDtypeStruct((NW * rows, 128), jnp.float32),
        mesh=plsc.VectorSubcoreMesh(core_axis_name="c", subcore_axis_name="s"),
        scratch_types=[
            pltpu.VMEM((n, CHUNK), jnp.int32),
            pltpu.VMEM((n, CHUNK), jnp.int32),
            pltpu.VMEM((CHUNK, 32), jnp.float32),
            pltpu.VMEM((CHUNK, 32), jnp.float32),
            pltpu.VMEM((CHUNK, 32), jnp.float32),
            pltpu.VMEM((CHUNK, 32), jnp.float32),
            pltpu.VMEM((32, 128), jnp.float32),
            pltpu.VMEM((32, 128), jnp.float32),
            pltpu.SemaphoreType.DMA,
            pltpu.SemaphoreType.DMA,
            pltpu.SemaphoreType.DMA,
            pltpu.SemaphoreType.DMA,
            pltpu.SemaphoreType.DMA,
            pltpu.SemaphoreType.DMA,
        ],
        compiler_params=pltpu.CompilerParams(use_tc_tiling_on_sc=False),
    )
    def dec_kernel(za_hbm, zb_hbm, s_hbm, d_hbm, out_hbm,
                   s_v, d_v, a0, b0, a1, b1, o0, o1,
                   ga0, gb0, ga1, gb1, sem_s0, sem_s1):
        c = lax.axis_index("c")
        s = lax.axis_index("s")
        wid = c * NS + s
        pltpu.sync_copy(s_hbm.at[pl.ds(k0 + wid * n, n)], s_v)
        pltpu.sync_copy(d_hbm.at[pl.ds(k0 + wid * n, n)], d_v)
        base4 = wid * rows

        def gath(j, a_v, b_v, sa, sb):
            pltpu.async_copy(za_hbm.at[s_v.at[j]], a_v, sa)
            pltpu.async_copy(zb_hbm.at[d_v.at[j]], b_v, sb)

        def gwait(j, a_v, b_v, sa, sb):
            pltpu.make_async_copy(za_hbm.at[s_v.at[j]], a_v, sa).wait()
            pltpu.make_async_copy(zb_hbm.at[d_v.at[j]], b_v, sb).wait()

        def relu_into(o_v, a_v, b_v):
            def rbody(q, carry2):
                for i in range(4):
                    for half in range(2):
                        si = pl.ds(16 * half, 16)
                        so = pl.ds(32 * i + 16 * half, 16)
                        o_v[q, so] = jnp.maximum(
                            a_v[4 * q + i, si] + b_v[4 * q + i, si], 0.0)
                return carry2

            lax.fori_loop(0, CHUNK // 4, rbody, 0)

        def store(j, o_v, sem):
            pltpu.async_copy(
                o_v, out_hbm.at[pl.ds(base4 + j * (CHUNK // 4), CHUNK // 4)],
                sem)

        def swait(j, o_v, sem):
            pltpu.make_async_copy(
                o_v, out_hbm.at[pl.ds(base4 + j * (CHUNK // 4), CHUNK // 4)],
                sem).wait()

        gath(0, a0, b0, ga0, gb0)
        gath(1, a1, b1, ga1, gb1)
        gwait(0, a0, b0, ga0, gb0)
        relu_into(o0, a0, b0)
        store(0, o0, sem_s0)
        gath(2, a0, b0, ga0, gb0)
        gwait(1, a1, b1, ga1, gb1)
        relu_into(o1, a1, b1)
        store(1, o1, sem_s1)
        gath(3, a1, b1, ga1, gb1)

        def pair(g, carry):
            j0 = 2 * g
            gwait(j0, a0, b0, ga0, gb0)
            swait(j0 - 2, o0, sem_s0)
            relu_into(o0, a0, b0)
            store(j0, o0, sem_s0)
            gath(j0 + 2, a0, b0, ga0, gb0)
            gwait(j0 + 1, a1, b1, ga1, gb1)
            swait(j0 - 1, o1, sem_s1)
            relu_into(o1, a1, b1)
            store(j0 + 1, o1, sem_s1)
            gath(j0 + 3, a1, b1, ga1, gb1)
            return carry

        if n % 2 == 1:
            lax.fori_loop(1, (n - 3) // 2, pair, 0)
            jm = n - 3
            gwait(jm, a0, b0, ga0, gb0)
            swait(jm - 2, o0, sem_s0)
            relu_into(o0, a0, b0)
            store(jm, o0, sem_s0)
            gath(jm + 2, a0, b0, ga0, gb0)
            gwait(jm + 1, a1, b1, ga1, gb1)
            swait(jm - 1, o1, sem_s1)
            relu_into(o1, a1, b1)
            store(jm + 1, o1, sem_s1)
            gwait(jm + 2, a0, b0, ga0, gb0)
            swait(jm, o0, sem_s0)
            relu_into(o0, a0, b0)
            store(jm + 2, o0, sem_s0)
            swait(jm + 1, o1, sem_s1)
            swait(jm + 2, o0, sem_s0)
        else:
            lax.fori_loop(1, (n - 2) // 2, pair, 0)
            jm = n - 2
            gwait(jm, a0, b0, ga0, gb0)
            swait(jm - 2, o0, sem_s0)
            relu_into(o0, a0, b0)
            store(jm, o0, sem_s0)
            gwait(jm + 1, a1, b1, ga1, gb1)
            swait(jm - 1, o1, sem_s1)
            relu_into(o1, a1, b1)
            store(jm + 1, o1, sem_s1)
            swait(jm, o0, sem_s0)
            swait(jm + 1, o1, sem_s1)

    return dec_kernel(za, zb, s_idx, d_idx)


_RB = 1000
_NBLK = N_NODES // _RB


def _tc_premul(x, W1l, W1r):
    def body(x_ref, wl_ref, wr_ref, p_ref, r_ref):
        xb = x_ref[...]
        p_ref[...] = jnp.dot(xb, wl_ref[...], preferred_element_type=jnp.float32)
        r_ref[...] = jnp.dot(xb, wr_ref[...], preferred_element_type=jnp.float32)

    return pl.pallas_call(
        body,
        grid=(_NBLK,),
        in_specs=[
            pl.BlockSpec((_RB, 128), lambda i: (i, 0)),
            pl.BlockSpec((128, 64), lambda i: (0, 0)),
            pl.BlockSpec((128, 64), lambda i: (0, 0)),
        ],
        out_specs=[
            pl.BlockSpec((_RB, 64), lambda i: (i, 0)),
            pl.BlockSpec((_RB, 64), lambda i: (i, 0)),
        ],
        out_shape=[
            jax.ShapeDtypeStruct((N_NODES, 64), jnp.float32),
            jax.ShapeDtypeStruct((N_NODES, 64), jnp.float32),
        ],
    )(x, W1l, W1r)


def _tc_layer2_premul(part64, part_deg, r1, b1, W2l, W2r):
    def body(p_ref, d_ref, r1_ref, b1_ref, wl_ref, wr_ref, p2_ref, r2_ref):
        agg = p_ref[0] + p_ref[1]
        deg = d_ref[0, :, 0:1] + d_ref[1, :, 0:1]
        invd = 1.0 / jnp.maximum(deg, 1.0)
        h = jnp.maximum(agg * invd + b1_ref[...] + r1_ref[...], 0.0)
        p2_ref[...] = jnp.dot(h, wl_ref[...], preferred_element_type=jnp.float32)
        r2_ref[...] = jnp.dot(h, wr_ref[...], preferred_element_type=jnp.float32)

    return pl.pallas_call(
        body,
        grid=(_NBLK,),
        in_specs=[
            pl.BlockSpec((NC, _RB, 64), lambda i: (0, i, 0)),
            pl.BlockSpec((NC, _RB, 16), lambda i: (0, i, 0)),
            pl.BlockSpec((_RB, 64), lambda i: (i, 0)),
            pl.BlockSpec((1, 64), lambda i: (0, 0)),
            pl.BlockSpec((64, 32), lambda i: (0, 0)),
            pl.BlockSpec((64, 32), lambda i: (0, 0)),
        ],
        out_specs=[
            pl.BlockSpec((_RB, 32), lambda i: (i, 0)),
            pl.BlockSpec((_RB, 32), lambda i: (i, 0)),
        ],
        out_shape=[
            jax.ShapeDtypeStruct((N_NODES, 32), jnp.float32),
            jax.ShapeDtypeStruct((N_NODES, 32), jnp.float32),
        ],
    )(part64, part_deg, r1, b1, W2l, W2r)


def _tc_decoder_tables(part32, part_deg, r2, b2, Wa_s, Wa_d, ba):
    def body(p_ref, d_ref, r2_ref, b2_ref, ws_ref, wd_ref, ba_ref,
             za_ref, zb_ref):
        agg = p_ref[0] + p_ref[1]
        deg = d_ref[0, :, 0:1] + d_ref[1, :, 0:1]
        invd = 1.0 / jnp.maximum(deg, 1.0)
        z = agg * invd + b2_ref[...] + r2_ref[...]
        za_ref[...] = (jnp.dot(z, ws_ref[...], preferred_element_type=jnp.float32)
                       + ba_ref[...])
        zb_ref[...] = jnp.dot(z, wd_ref[...], preferred_element_type=jnp.float32)

    return pl.pallas_call(
        body,
        grid=(_NBLK,),
        in_specs=[
            pl.BlockSpec((NC, _RB, 32), lambda i: (0, i, 0)),
            pl.BlockSpec((NC, _RB, 16), lambda i: (0, i, 0)),
            pl.BlockSpec((_RB, 32), lambda i: (i, 0)),
            pl.BlockSpec((1, 32), lambda i: (0, 0)),
            pl.BlockSpec((32, 32), lambda i: (0, 0)),
            pl.BlockSpec((32, 32), lambda i: (0, 0)),
            pl.BlockSpec((1, 32), lambda i: (0, 0)),
        ],
        out_specs=[
            pl.BlockSpec((_RB, 32), lambda i: (i, 0)),
            pl.BlockSpec((_RB, 32), lambda i: (i, 0)),
        ],
        out_shape=[
            jax.ShapeDtypeStruct((N_NODES, 32), jnp.float32),
            jax.ShapeDtypeStruct((N_NODES, 32), jnp.float32),
        ],
    )(part32, part_deg, r2, b2, Wa_s, Wa_d, ba)


_RB4 = 1024


def _tc_mlp(h1p, Wb4, bb4, Wc4, bc4):
    def body(h_ref, wb_ref, bb_ref, wc_ref, bc_ref, f_ref, a_ref, o_ref):
        h2 = jnp.maximum(
            jnp.dot(h_ref[...], wb_ref[...], preferred_element_type=jnp.float32)
            + bb_ref[...], 0.0)
        g = (jnp.dot(h2, wc_ref[...], preferred_element_type=jnp.float32)
             + bc_ref[...])
        t = jnp.dot(g, f_ref[...], preferred_element_type=jnp.float32)
        lane = lax.broadcasted_iota(jnp.int32, (_RB4, 128), 1)
        row = lax.broadcasted_iota(jnp.int32, (_RB4, 128), 0)
        sel = (lane // 4 == row % 32).astype(jnp.float32)
        o_ref[...] = jnp.dot(a_ref[...], t * sel,
                             preferred_element_type=jnp.float32)

    nrows = h1p.shape[0]
    assert nrows % _RB4 == 0
    lane4 = jnp.arange(128, dtype=jnp.int32) % 4
    Fsp = (lane4[None, :] == jnp.arange(4, dtype=jnp.int32)[:, None]
           ).astype(jnp.float32)
    Asum = (jnp.arange(_RB4, dtype=jnp.int32)[None, :] // 32
            == jnp.arange(_RB4 // 32, dtype=jnp.int32)[:, None]
            ).astype(jnp.float32)
    return pl.pallas_call(
        body,
        grid=(nrows // _RB4,),
        in_specs=[
            pl.BlockSpec((_RB4, 128), lambda i: (i, 0)),
            pl.BlockSpec((128, 64), lambda i: (0, 0)),
            pl.BlockSpec((1, 64), lambda i: (0, 0)),
            pl.BlockSpec((64, 4), lambda i: (0, 0)),
            pl.BlockSpec((1, 4), lambda i: (0, 0)),
            pl.BlockSpec((4, 128), lambda i: (0, 0)),
            pl.BlockSpec((_RB4 // 32, _RB4), lambda i: (0, 0)),
        ],
        out_specs=pl.BlockSpec((_RB4 // 32, 128), lambda i: (i, 0)),
        out_shape=jax.ShapeDtypeStruct((nrows // 32, 128), jnp.float32),
    )(h1p, Wb4, bb4, Wc4, bc4, Fsp, Asum)


def kernel(x, edge_index, edge_label_index, W1l, b1, W1r, W2l, b2, W2r,
           Wa, ba, Wb, bb, Wc, bc):
    src = _pad_idx(edge_index[0], 0)
    dst = _pad_idx(edge_index[1], N_NODES, spread=True)
    ls = _pad_idx(edge_label_index[0], 0)
    ld = _pad_idx(edge_label_index[1], 0)

    b1r = b1.reshape(1, 64)
    b2r = b2.reshape(1, 32)
    bar = ba.reshape(1, 32)
    z128 = jnp.zeros((32, 16), jnp.float32)
    z64 = jnp.zeros((16, 1), jnp.float32)
    Wb4 = jnp.block([[Wb if i == j else z128 for j in range(4)]
                     for i in range(4)])
    Wc4 = jnp.block([[Wc if i == j else z64 for j in range(4)]
                     for i in range(4)])
    bb4 = jnp.tile(bb, 4).reshape(1, 64)
    bc4 = jnp.tile(bc, 4).reshape(1, 4)

    p1, r1 = _tc_premul(x, W1l, W1r)
    part64, part_deg = _sc_aggregate(p1, src, dst, 64, with_deg=True)
    p2, r2 = _tc_layer2_premul(part64, part_deg, r1, b1r, W2l, W2r)
    (part32,) = _sc_aggregate(p2, src, dst, 32, with_deg=False)
    za, zb = _tc_decoder_tables(part32, part_deg, r2, b2r,
                                Wa[:32], Wa[32:], bar)
    outs = []
    k0 = 0
    for n in (27, 26, 26):
        h1s = _sc_decoder_gather(za, zb, ls, ld, k0, n)
        outs.append(_tc_mlp(h1s, Wb4, bb4, Wc4, bc4))
        k0 += NW * n
    o = jnp.concatenate(outs, axis=0)
    return o[:N_PRED // 128].reshape(N_PRED)


# speedup vs baseline: 1.6185x; 1.3862x over previous
import functools

import jax
import jax.numpy as jnp
from jax import lax
from jax.experimental import pallas as pl
from jax.experimental.pallas import tpu as pltpu
from jax.experimental.pallas import tpu_sc as plsc

N_NODES = 10000
N_EDGES = 320000
N_PRED = 320000

NC = 2
NS = 16
NW = NC * NS
CHUNK = 128
EW = N_EDGES // NW
KCH = -(-EW // CHUNK)
EWP = KCH * CHUNK
EP = NW * EWP
NPAD = 10112
RPS = NPAD // NS


NCH = EP // CHUNK


def _pad_idx(a, pad_base, spread_mod):
    a = a.astype(jnp.int32)
    npad = EP - a.shape[0]
    pad = pad_base + jnp.arange(npad, dtype=jnp.int32) % spread_mod
    return jnp.concatenate([a, pad]).reshape(NCH, CHUNK)


def _sc_aggregate(table, src_idx, dst_idx, width, with_deg):
    assert KCH % 2 == 1 and KCH >= 3
    zeros_w = jnp.zeros((NPAD, width), dtype=jnp.float32)
    out_type = [jax.ShapeDtypeStruct((NC, NPAD, width), jnp.float32)]
    scratch = [
        pltpu.VMEM((KCH, CHUNK), jnp.int32),
        pltpu.VMEM((KCH, CHUNK), jnp.int32),
        pltpu.VMEM((CHUNK, width), jnp.float32),
        pltpu.VMEM((CHUNK, width), jnp.float32),
        pltpu.VMEM_SHARED((NPAD, width), jnp.float32),
        pltpu.SemaphoreType.DMA,
        pltpu.SemaphoreType.DMA,
    ]
    extra_in = ()
    if with_deg:
        zeros_d = jnp.zeros((NPAD, 16), dtype=jnp.float32)
        ones = jnp.ones((CHUNK, 16), dtype=jnp.float32)
        extra_in = (zeros_d, ones)
        out_type.append(jax.ShapeDtypeStruct((NC, NPAD, 16), jnp.float32))
        scratch.extend([
            pltpu.VMEM((CHUNK, 16), jnp.float32),
            pltpu.VMEM_SHARED((NPAD, 16), jnp.float32),
        ])

    @functools.partial(
        pl.kernel,
        out_type=tuple(out_type),
        mesh=plsc.VectorSubcoreMesh(core_axis_name="c", subcore_axis_name="s"),
        scratch_types=scratch,
        compiler_params=pltpu.CompilerParams(use_tc_tiling_on_sc=False),
    )
    def agg_kernel(table_hbm, src_hbm, dst_hbm, zeros_hbm, *rest):
        if with_deg:
            (zerosd_hbm, ones_hbm, out_hbm, outd_hbm,
             src_v, dst_v, rows0, rows1, acc_s, sem0, sem1,
             ones_v, accd_s) = rest
        else:
            (out_hbm, src_v, dst_v, rows0, rows1, acc_s,
             sem0, sem1) = rest
        c = lax.axis_index("c")
        s = lax.axis_index("s")
        wid = c * NS + s
        pltpu.sync_copy(zeros_hbm.at[pl.ds(s * RPS, RPS)],
                        acc_s.at[pl.ds(s * RPS, RPS)])
        pltpu.sync_copy(src_hbm.at[pl.ds(wid * KCH, KCH)], src_v)
        pltpu.sync_copy(dst_hbm.at[pl.ds(wid * KCH, KCH)], dst_v)
        if with_deg:
            pltpu.sync_copy(zerosd_hbm.at[pl.ds(s * RPS, RPS)],
                            accd_s.at[pl.ds(s * RPS, RPS)])
            pltpu.sync_copy(ones_hbm, ones_v)
        plsc.subcore_barrier()

        def scat(rows_v, j):
            pltpu.sync_copy(rows_v, acc_s.at[dst_v.at[j]], add=True)
            if with_deg:
                pltpu.sync_copy(ones_v, accd_s.at[dst_v.at[j]], add=True)

        pltpu.async_copy(table_hbm.at[src_v.at[0]], rows0, sem0)

        def pair(g, carry):
            j0 = 2 * g
            pltpu.async_copy(table_hbm.at[src_v.at[j0 + 1]], rows1, sem1)
            pltpu.make_async_copy(table_hbm.at[src_v.at[j0]],
                                  rows0, sem0).wait()
            scat(rows0, j0)
            pltpu.async_copy(table_hbm.at[src_v.at[j0 + 2]], rows0, sem0)
            pltpu.make_async_copy(table_hbm.at[src_v.at[j0 + 1]],
                                  rows1, sem1).wait()
            scat(rows1, j0 + 1)
            return carry

        lax.fori_loop(0, KCH // 2, pair, 0)
        pltpu.make_async_copy(table_hbm.at[src_v.at[KCH - 1]],
                              rows0, sem0).wait()
        scat(rows0, KCH - 1)
        plsc.subcore_barrier()
        pltpu.sync_copy(acc_s.at[pl.ds(s * RPS, RPS)],
                        out_hbm.at[c, pl.ds(s * RPS, RPS)])
        if with_deg:
            pltpu.sync_copy(accd_s.at[pl.ds(s * RPS, RPS)],
                            outd_hbm.at[c, pl.ds(s * RPS, RPS)])

    return agg_kernel(table, src_idx, dst_idx, zeros_w, *extra_in)


def _sc_decoder_gather(za, zb, s_idx, d_idx, k0, n):
    assert n >= 4
    rows = n * (CHUNK // 4)

    @functools.partial(
        pl.kernel,
        out_type=jax.ShapeDtypeStruct((NW * rows, 128), jnp.float32),
        mesh=plsc.VectorSubcoreMesh(core_axis_name="c", subcore_axis_name="s"),
        scratch_types=[
            pltpu.VMEM((n, CHUNK), jnp.int32),
            pltpu.VMEM((n, CHUNK), jnp.int32),
            pltpu.VMEM((CHUNK, 32), jnp.float32),
            pltpu.VMEM((CHUNK, 32), jnp.float32),
            pltpu.VMEM((CHUNK, 32), jnp.float32),
            pltpu.VMEM((CHUNK, 32), jnp.float32),
            pltpu.VMEM((32, 128), jnp.float32),
            pltpu.VMEM((32, 128), jnp.float32),
            pltpu.SemaphoreType.DMA,
            pltpu.SemaphoreType.DMA,
            pltpu.SemaphoreType.DMA,
            pltpu.SemaphoreType.DMA,
            pltpu.SemaphoreType.DMA,
            pltpu.SemaphoreType.DMA,
        ],
        compiler_params=pltpu.CompilerParams(use_tc_tiling_on_sc=False),
    )
    def dec_kernel(za_hbm, zb_hbm, s_hbm, d_hbm, out_hbm,
                   s_v, d_v, a0, b0, a1, b1, o0, o1,
                   ga0, gb0, ga1, gb1, sem_s0, sem_s1):
        c = lax.axis_index("c")
        s = lax.axis_index("s")
        wid = c * NS + s
        pltpu.sync_copy(s_hbm.at[pl.ds(k0 + wid * n, n)], s_v)
        pltpu.sync_copy(d_hbm.at[pl.ds(k0 + wid * n, n)], d_v)
        base4 = wid * rows

        def gath(j, a_v, b_v, sa, sb):
            pltpu.async_copy(za_hbm.at[s_v.at[j]], a_v, sa)
            pltpu.async_copy(zb_hbm.at[d_v.at[j]], b_v, sb)

        def gwait(j, a_v, b_v, sa, sb):
            pltpu.make_async_copy(za_hbm.at[s_v.at[j]], a_v, sa).wait()
            pltpu.make_async_copy(zb_hbm.at[d_v.at[j]], b_v, sb).wait()

        def relu_into(o_v, a_v, b_v):
            def rbody(q, carry2):
                for i in range(4):
                    for half in range(2):
                        si = pl.ds(16 * half, 16)
                        so = pl.ds(32 * i + 16 * half, 16)
                        o_v[q, so] = jnp.maximum(
                            a_v[4 * q + i, si] + b_v[4 * q + i, si], 0.0)
                return carry2

            lax.fori_loop(0, CHUNK // 4, rbody, 0)

        def store(j, o_v, sem):
            pltpu.async_copy(
                o_v, out_hbm.at[pl.ds(base4 + j * (CHUNK // 4), CHUNK // 4)],
                sem)

        def swait(j, o_v, sem):
            pltpu.make_async_copy(
                o_v, out_hbm.at[pl.ds(base4 + j * (CHUNK // 4), CHUNK // 4)],
                sem).wait()

        gath(0, a0, b0, ga0, gb0)
        gath(1, a1, b1, ga1, gb1)
        gwait(0, a0, b0, ga0, gb0)
        relu_into(o0, a0, b0)
        store(0, o0, sem_s0)
        gath(2, a0, b0, ga0, gb0)
        gwait(1, a1, b1, ga1, gb1)
        relu_into(o1, a1, b1)
        store(1, o1, sem_s1)
        gath(3, a1, b1, ga1, gb1)

        def pair(g, carry):
            j0 = 2 * g
            gwait(j0, a0, b0, ga0, gb0)
            swait(j0 - 2, o0, sem_s0)
            relu_into(o0, a0, b0)
            store(j0, o0, sem_s0)
            gath(j0 + 2, a0, b0, ga0, gb0)
            gwait(j0 + 1, a1, b1, ga1, gb1)
            swait(j0 - 1, o1, sem_s1)
            relu_into(o1, a1, b1)
            store(j0 + 1, o1, sem_s1)
            gath(j0 + 3, a1, b1, ga1, gb1)
            return carry

        if n % 2 == 1:
            lax.fori_loop(1, (n - 3) // 2, pair, 0)
            jm = n - 3
            gwait(jm, a0, b0, ga0, gb0)
            swait(jm - 2, o0, sem_s0)
            relu_into(o0, a0, b0)
            store(jm, o0, sem_s0)
            gath(jm + 2, a0, b0, ga0, gb0)
            gwait(jm + 1, a1, b1, ga1, gb1)
            swait(jm - 1, o1, sem_s1)
            relu_into(o1, a1, b1)
            store(jm + 1, o1, sem_s1)
            gwait(jm + 2, a0, b0, ga0, gb0)
            swait(jm, o0, sem_s0)
            relu_into(o0, a0, b0)
            store(jm + 2, o0, sem_s0)
            swait(jm + 1, o1, sem_s1)
            swait(jm + 2, o0, sem_s0)
        else:
            lax.fori_loop(1, (n - 2) // 2, pair, 0)
            jm = n - 2
            gwait(jm, a0, b0, ga0, gb0)
            swait(jm - 2, o0, sem_s0)
            relu_into(o0, a0, b0)
            store(jm, o0, sem_s0)
            gwait(jm + 1, a1, b1, ga1, gb1)
            swait(jm - 1, o1, sem_s1)
            relu_into(o1, a1, b1)
            store(jm + 1, o1, sem_s1)
            swait(jm, o0, sem_s0)
            swait(jm + 1, o1, sem_s1)

    return dec_kernel(za, zb, s_idx, d_idx)


_RB = 1000
_NBLK = N_NODES // _RB


def _tc_premul(x, W1l, W1r):
    def body(x_ref, wl_ref, wr_ref, p_ref, r_ref):
        xb = x_ref[...]
        p_ref[...] = jnp.dot(xb, wl_ref[...], preferred_element_type=jnp.float32)
        r_ref[...] = jnp.dot(xb, wr_ref[...], preferred_element_type=jnp.float32)

    return pl.pallas_call(
        body,
        grid=(_NBLK,),
        in_specs=[
            pl.BlockSpec((_RB, 128), lambda i: (i, 0)),
            pl.BlockSpec((128, 64), lambda i: (0, 0)),
            pl.BlockSpec((128, 64), lambda i: (0, 0)),
        ],
        out_specs=[
            pl.BlockSpec((_RB, 64), lambda i: (i, 0)),
            pl.BlockSpec((_RB, 64), lambda i: (i, 0)),
        ],
        out_shape=[
            jax.ShapeDtypeStruct((N_NODES, 64), jnp.float32),
            jax.ShapeDtypeStruct((N_NODES, 64), jnp.float32),
        ],
    )(x, W1l, W1r)


def _tc_layer2_premul(part64, part_deg, r1, b1, W2l, W2r):
    def body(p_ref, d_ref, r1_ref, b1_ref, wl_ref, wr_ref, p2_ref, r2_ref):
        agg = p_ref[0] + p_ref[1]
        deg = d_ref[0, :, 0:1] + d_ref[1, :, 0:1]
        invd = 1.0 / jnp.maximum(deg, 1.0)
        h = jnp.maximum(agg * invd + b1_ref[...] + r1_ref[...], 0.0)
        p2_ref[...] = jnp.dot(h, wl_ref[...], preferred_element_type=jnp.float32)
        r2_ref[...] = jnp.dot(h, wr_ref[...], preferred_element_type=jnp.float32)

    return pl.pallas_call(
        body,
        grid=(_NBLK,),
        in_specs=[
            pl.BlockSpec((NC, _RB, 64), lambda i: (0, i, 0)),
            pl.BlockSpec((NC, _RB, 16), lambda i: (0, i, 0)),
            pl.BlockSpec((_RB, 64), lambda i: (i, 0)),
            pl.BlockSpec((1, 64), lambda i: (0, 0)),
            pl.BlockSpec((64, 32), lambda i: (0, 0)),
            pl.BlockSpec((64, 32), lambda i: (0, 0)),
        ],
        out_specs=[
            pl.BlockSpec((_RB, 32), lambda i: (i, 0)),
            pl.BlockSpec((_RB, 32), lambda i: (i, 0)),
        ],
        out_shape=[
            jax.ShapeDtypeStruct((N_NODES, 32), jnp.float32),
            jax.ShapeDtypeStruct((N_NODES, 32), jnp.float32),
        ],
    )(part64, part_deg, r1, b1, W2l, W2r)


def _tc_decoder_tables(part32, part_deg, r2, b2, Wa_s, Wa_d, ba):
    def body(p_ref, d_ref, r2_ref, b2_ref, ws_ref, wd_ref, ba_ref,
             za_ref, zb_ref):
        agg = p_ref[0] + p_ref[1]
        deg = d_ref[0, :, 0:1] + d_ref[1, :, 0:1]
        invd = 1.0 / jnp.maximum(deg, 1.0)
        z = agg * invd + b2_ref[...] + r2_ref[...]
        za_ref[...] = (jnp.dot(z, ws_ref[...], preferred_element_type=jnp.float32)
                       + ba_ref[...])
        zb_ref[...] = jnp.dot(z, wd_ref[...], preferred_element_type=jnp.float32)

    return pl.pallas_call(
        body,
        grid=(_NBLK,),
        in_specs=[
            pl.BlockSpec((NC, _RB, 32), lambda i: (0, i, 0)),
            pl.BlockSpec((NC, _RB, 16), lambda i: (0, i, 0)),
            pl.BlockSpec((_RB, 32), lambda i: (i, 0)),
            pl.BlockSpec((1, 32), lambda i: (0, 0)),
            pl.BlockSpec((32, 32), lambda i: (0, 0)),
            pl.BlockSpec((32, 32), lambda i: (0, 0)),
            pl.BlockSpec((1, 32), lambda i: (0, 0)),
        ],
        out_specs=[
            pl.BlockSpec((_RB, 32), lambda i: (i, 0)),
            pl.BlockSpec((_RB, 32), lambda i: (i, 0)),
        ],
        out_shape=[
            jax.ShapeDtypeStruct((N_NODES, 32), jnp.float32),
            jax.ShapeDtypeStruct((N_NODES, 32), jnp.float32),
        ],
    )(part32, part_deg, r2, b2, Wa_s, Wa_d, ba)


_RB4 = 1024


def _tc_mlp(h1p, Wb4, bb4, Wc4, bc4):
    def body(h_ref, wb_ref, bb_ref, wc_ref, bc_ref, f_ref, a_ref, o_ref):
        h2 = jnp.maximum(
            jnp.dot(h_ref[...], wb_ref[...], preferred_element_type=jnp.float32)
            + bb_ref[...], 0.0)
        g = (jnp.dot(h2, wc_ref[...], preferred_element_type=jnp.float32)
             + bc_ref[...])
        t = jnp.dot(g, f_ref[...], preferred_element_type=jnp.float32)
        lane = lax.broadcasted_iota(jnp.int32, (_RB4, 128), 1)
        row = lax.broadcasted_iota(jnp.int32, (_RB4, 128), 0)
        sel = (lane // 4 == row % 32).astype(jnp.float32)
        o_ref[...] = jnp.dot(a_ref[...], t * sel,
                             preferred_element_type=jnp.float32)

    nrows = h1p.shape[0]
    assert nrows % _RB4 == 0
    lane4 = jnp.arange(128, dtype=jnp.int32) % 4
    Fsp = (lane4[None, :] == jnp.arange(4, dtype=jnp.int32)[:, None]
           ).astype(jnp.float32)
    Asum = (jnp.arange(_RB4, dtype=jnp.int32)[None, :] // 32
            == jnp.arange(_RB4 // 32, dtype=jnp.int32)[:, None]
            ).astype(jnp.float32)
    return pl.pallas_call(
        body,
        grid=(nrows // _RB4,),
        in_specs=[
            pl.BlockSpec((_RB4, 128), lambda i: (i, 0)),
            pl.BlockSpec((128, 64), lambda i: (0, 0)),
            pl.BlockSpec((1, 64), lambda i: (0, 0)),
            pl.BlockSpec((64, 4), lambda i: (0, 0)),
            pl.BlockSpec((1, 4), lambda i: (0, 0)),
            pl.BlockSpec((4, 128), lambda i: (0, 0)),
            pl.BlockSpec((_RB4 // 32, _RB4), lambda i: (0, 0)),
        ],
        out_specs=pl.BlockSpec((_RB4 // 32, 128), lambda i: (i, 0)),
        out_shape=jax.ShapeDtypeStruct((nrows // 32, 128), jnp.float32),
    )(h1p, Wb4, bb4, Wc4, bc4, Fsp, Asum)


def kernel(x, edge_index, edge_label_index, W1l, b1, W1r, W2l, b2, W2r,
           Wa, ba, Wb, bb, Wc, bc):
    src = _pad_idx(edge_index[0], 0, N_NODES)
    dst = _pad_idx(edge_index[1], N_NODES, NPAD - N_NODES)
    ls = _pad_idx(edge_label_index[0], 0, N_NODES)
    ld = _pad_idx(edge_label_index[1], 0, N_NODES)

    b1r = b1.reshape(1, 64)
    b2r = b2.reshape(1, 32)
    bar = ba.reshape(1, 32)
    z128 = jnp.zeros((32, 16), jnp.float32)
    z64 = jnp.zeros((16, 1), jnp.float32)
    Wb4 = jnp.block([[Wb if i == j else z128 for j in range(4)]
                     for i in range(4)])
    Wc4 = jnp.block([[Wc if i == j else z64 for j in range(4)]
                     for i in range(4)])
    bb4 = jnp.tile(bb, 4).reshape(1, 64)
    bc4 = jnp.tile(bc, 4).reshape(1, 4)

    p1, r1 = _tc_premul(x, W1l, W1r)
    part64, part_deg = _sc_aggregate(p1, src, dst, 64, with_deg=True)
    p2, r2 = _tc_layer2_premul(part64, part_deg, r1, b1r, W2l, W2r)
    (part32,) = _sc_aggregate(p2, src, dst, 32, with_deg=False)
    za, zb = _tc_decoder_tables(part32, part_deg, r2, b2r,
                                Wa[:32], Wa[32:], bar)
    outs = []
    k0 = 0
    for n in (20, 20, 20, 19):
        h1s = _sc_decoder_gather(za, zb, ls, ld, k0, n)
        outs.append(_tc_mlp(h1s, Wb4, bb4, Wc4, bc4))
        k0 += NW * n
    o = jnp.concatenate(outs, axis=0)
    return o[:N_PRED // 128].reshape(N_PRED)


# speedup vs baseline: 1.6187x; 1.0001x over previous
import functools

import jax
import jax.numpy as jnp
from jax import lax
from jax.experimental import pallas as pl
from jax.experimental.pallas import tpu as pltpu
from jax.experimental.pallas import tpu_sc as plsc

N_NODES = 10000
N_EDGES = 320000
N_PRED = 320000

NC = 2
NS = 16
NW = NC * NS
CHUNK = 128
EW = N_EDGES // NW
KCH = -(-EW // CHUNK)
EWP = KCH * CHUNK
EP = NW * EWP
NPAD = 10112
RPS = NPAD // NS


NCH = EP // CHUNK


def _pad_idx(a, pad_base, spread_mod):
    a = a.astype(jnp.int32)
    npad = EP - a.shape[0]
    pad = pad_base + jnp.arange(npad, dtype=jnp.int32) % spread_mod
    return jnp.concatenate([a, pad]).reshape(NCH, CHUNK)


def _sc_aggregate(table, src_idx, dst_idx, width, with_deg):
    assert KCH % 2 == 1 and KCH >= 3
    zeros_w = jnp.zeros((NPAD, width), dtype=jnp.float32)
    out_type = [jax.ShapeDtypeStruct((NC, NPAD, width), jnp.float32)]
    scratch = [
        pltpu.VMEM((KCH, CHUNK), jnp.int32),
        pltpu.VMEM((KCH, CHUNK), jnp.int32),
        pltpu.VMEM((CHUNK, width), jnp.float32),
        pltpu.VMEM((CHUNK, width), jnp.float32),
        pltpu.VMEM_SHARED((NPAD, width), jnp.float32),
        pltpu.SemaphoreType.DMA,
        pltpu.SemaphoreType.DMA,
    ]
    extra_in = ()
    if with_deg:
        zeros_d = jnp.zeros((NPAD, 16), dtype=jnp.float32)
        ones = jnp.ones((CHUNK, 16), dtype=jnp.float32)
        extra_in = (zeros_d, ones)
        out_type.append(jax.ShapeDtypeStruct((NC, NPAD, 16), jnp.float32))
        scratch.extend([
            pltpu.VMEM((CHUNK, 16), jnp.float32),
            pltpu.VMEM_SHARED((NPAD, 16), jnp.float32),
        ])

    @functools.partial(
        pl.kernel,
        out_type=tuple(out_type),
        mesh=plsc.VectorSubcoreMesh(core_axis_name="c", subcore_axis_name="s"),
        scratch_types=scratch,
        compiler_params=pltpu.CompilerParams(use_tc_tiling_on_sc=False),
    )
    def agg_kernel(table_hbm, src_hbm, dst_hbm, zeros_hbm, *rest):
        if with_deg:
            (zerosd_hbm, ones_hbm, out_hbm, outd_hbm,
             src_v, dst_v, rows0, rows1, acc_s, sem0, sem1,
             ones_v, accd_s) = rest
        else:
            (out_hbm, src_v, dst_v, rows0, rows1, acc_s,
             sem0, sem1) = rest
        c = lax.axis_index("c")
        s = lax.axis_index("s")
        wid = c * NS + s
        pltpu.sync_copy(zeros_hbm.at[pl.ds(s * RPS, RPS)],
                        acc_s.at[pl.ds(s * RPS, RPS)])
        pltpu.sync_copy(src_hbm.at[pl.ds(wid * KCH, KCH)], src_v)
        pltpu.sync_copy(dst_hbm.at[pl.ds(wid * KCH, KCH)], dst_v)
        if with_deg:
            pltpu.sync_copy(zerosd_hbm.at[pl.ds(s * RPS, RPS)],
                            accd_s.at[pl.ds(s * RPS, RPS)])
            pltpu.sync_copy(ones_hbm, ones_v)
        plsc.subcore_barrier()

        def scat(rows_v, j):
            pltpu.sync_copy(rows_v, acc_s.at[dst_v.at[j]], add=True)
            if with_deg:
                pltpu.sync_copy(ones_v, accd_s.at[dst_v.at[j]], add=True)

        pltpu.async_copy(table_hbm.at[src_v.at[0]], rows0, sem0)

        def pair(g, carry):
            j0 = 2 * g
            pltpu.async_copy(table_hbm.at[src_v.at[j0 + 1]], rows1, sem1)
            pltpu.make_async_copy(table_hbm.at[src_v.at[j0]],
                                  rows0, sem0).wait()
            scat(rows0, j0)
            pltpu.async_copy(table_hbm.at[src_v.at[j0 + 2]], rows0, sem0)
            pltpu.make_async_copy(table_hbm.at[src_v.at[j0 + 1]],
                                  rows1, sem1).wait()
            scat(rows1, j0 + 1)
            return carry

        lax.fori_loop(0, KCH // 2, pair, 0)
        pltpu.make_async_copy(table_hbm.at[src_v.at[KCH - 1]],
                              rows0, sem0).wait()
        scat(rows0, KCH - 1)
        plsc.subcore_barrier()
        pltpu.sync_copy(acc_s.at[pl.ds(s * RPS, RPS)],
                        out_hbm.at[c, pl.ds(s * RPS, RPS)])
        if with_deg:
            pltpu.sync_copy(accd_s.at[pl.ds(s * RPS, RPS)],
                            outd_hbm.at[c, pl.ds(s * RPS, RPS)])

    return agg_kernel(table, src_idx, dst_idx, zeros_w, *extra_in)


def _sc_decoder_gather(za, zb, s_idx, d_idx, k0, n):
    assert n >= 4
    rows = n * (CHUNK // 4)

    @functools.partial(
        pl.kernel,
        out_type=jax.ShapeDtypeStruct((NW * rows, 128), jnp.float32),
        mesh=plsc.VectorSubcoreMesh(core_axis_name="c", subcore_axis_name="s"),
        scratch_types=[
            pltpu.VMEM((n, CHUNK), jnp.int32),
            pltpu.VMEM((n, CHUNK), jnp.int32),
            pltpu.VMEM((CHUNK, 32), jnp.float32),
            pltpu.VMEM((CHUNK, 32), jnp.float32),
            pltpu.VMEM((CHUNK, 32), jnp.float32),
            pltpu.VMEM((CHUNK, 32), jnp.float32),
            pltpu.VMEM((32, 128), jnp.float32),
            pltpu.VMEM((32, 128), jnp.float32),
            pltpu.SemaphoreType.DMA,
            pltpu.SemaphoreType.DMA,
            pltpu.SemaphoreType.DMA,
            pltpu.SemaphoreType.DMA,
            pltpu.SemaphoreType.DMA,
            pltpu.SemaphoreType.DMA,
        ],
        compiler_params=pltpu.CompilerParams(use_tc_tiling_on_sc=False),
    )
    def dec_kernel(za_hbm, zb_hbm, s_hbm, d_hbm, out_hbm,
                   s_v, d_v, a0, b0, a1, b1, o0, o1,
                   ga0, gb0, ga1, gb1, sem_s0, sem_s1):
        c = lax.axis_index("c")
        s = lax.axis_index("s")
        wid = c * NS + s
        pltpu.sync_copy(s_hbm.at[pl.ds(k0 + wid * n, n)], s_v)
        pltpu.sync_copy(d_hbm.at[pl.ds(k0 + wid * n, n)], d_v)
        base4 = wid * rows

        def gath(j, a_v, b_v, sa, sb):
            pltpu.async_copy(za_hbm.at[s_v.at[j]], a_v, sa)
            pltpu.async_copy(zb_hbm.at[d_v.at[j]], b_v, sb)

        def gwait(j, a_v, b_v, sa, sb):
            pltpu.make_async_copy(za_hbm.at[s_v.at[j]], a_v, sa).wait()
            pltpu.make_async_copy(zb_hbm.at[d_v.at[j]], b_v, sb).wait()

        def relu_into(o_v, a_v, b_v):
            def rbody(q, carry2):
                for i in range(4):
                    for half in range(2):
                        si = pl.ds(16 * half, 16)
                        so = pl.ds(32 * i + 16 * half, 16)
                        o_v[q, so] = jnp.maximum(
                            a_v[4 * q + i, si] + b_v[4 * q + i, si], 0.0)
                return carry2

            lax.fori_loop(0, CHUNK // 4, rbody, 0)

        def store(j, o_v, sem):
            pltpu.async_copy(
                o_v, out_hbm.at[pl.ds(base4 + j * (CHUNK // 4), CHUNK // 4)],
                sem)

        def swait(j, o_v, sem):
            pltpu.make_async_copy(
                o_v, out_hbm.at[pl.ds(base4 + j * (CHUNK // 4), CHUNK // 4)],
                sem).wait()

        gath(0, a0, b0, ga0, gb0)
        gath(1, a1, b1, ga1, gb1)
        gwait(0, a0, b0, ga0, gb0)
        relu_into(o0, a0, b0)
        store(0, o0, sem_s0)
        gath(2, a0, b0, ga0, gb0)
        gwait(1, a1, b1, ga1, gb1)
        relu_into(o1, a1, b1)
        store(1, o1, sem_s1)
        gath(3, a1, b1, ga1, gb1)

        def pair(g, carry):
            j0 = 2 * g
            gwait(j0, a0, b0, ga0, gb0)
            swait(j0 - 2, o0, sem_s0)
            relu_into(o0, a0, b0)
            store(j0, o0, sem_s0)
            gath(j0 + 2, a0, b0, ga0, gb0)
            gwait(j0 + 1, a1, b1, ga1, gb1)
            swait(j0 - 1, o1, sem_s1)
            relu_into(o1, a1, b1)
            store(j0 + 1, o1, sem_s1)
            gath(j0 + 3, a1, b1, ga1, gb1)
            return carry

        if n % 2 == 1:
            lax.fori_loop(1, (n - 3) // 2, pair, 0)
            jm = n - 3
            gwait(jm, a0, b0, ga0, gb0)
            swait(jm - 2, o0, sem_s0)
            relu_into(o0, a0, b0)
            store(jm, o0, sem_s0)
            gath(jm + 2, a0, b0, ga0, gb0)
            gwait(jm + 1, a1, b1, ga1, gb1)
            swait(jm - 1, o1, sem_s1)
            relu_into(o1, a1, b1)
            store(jm + 1, o1, sem_s1)
            gwait(jm + 2, a0, b0, ga0, gb0)
            swait(jm, o0, sem_s0)
            relu_into(o0, a0, b0)
            store(jm + 2, o0, sem_s0)
            swait(jm + 1, o1, sem_s1)
            swait(jm + 2, o0, sem_s0)
        else:
            lax.fori_loop(1, (n - 2) // 2, pair, 0)
            jm = n - 2
            gwait(jm, a0, b0, ga0, gb0)
            swait(jm - 2, o0, sem_s0)
            relu_into(o0, a0, b0)
            store(jm, o0, sem_s0)
            gwait(jm + 1, a1, b1, ga1, gb1)
            swait(jm - 1, o1, sem_s1)
            relu_into(o1, a1, b1)
            store(jm + 1, o1, sem_s1)
            swait(jm, o0, sem_s0)
            swait(jm + 1, o1, sem_s1)

    return dec_kernel(za, zb, s_idx, d_idx)


_RB = 1000
_NBLK = N_NODES // _RB


def _tc_premul(x, W1l, W1r):
    def body(x_ref, wl_ref, wr_ref, p_ref, r_ref):
        xb = x_ref[...]
        p_ref[...] = jnp.dot(xb, wl_ref[...], preferred_element_type=jnp.float32)
        r_ref[...] = jnp.dot(xb, wr_ref[...], preferred_element_type=jnp.float32)

    return pl.pallas_call(
        body,
        grid=(_NBLK,),
        in_specs=[
            pl.BlockSpec((_RB, 128), lambda i: (i, 0)),
            pl.BlockSpec((128, 64), lambda i: (0, 0)),
            pl.BlockSpec((128, 64), lambda i: (0, 0)),
        ],
        out_specs=[
            pl.BlockSpec((_RB, 64), lambda i: (i, 0)),
            pl.BlockSpec((_RB, 64), lambda i: (i, 0)),
        ],
        out_shape=[
            jax.ShapeDtypeStruct((N_NODES, 64), jnp.float32),
            jax.ShapeDtypeStruct((N_NODES, 64), jnp.float32),
        ],
    )(x, W1l, W1r)


def _tc_layer2_premul(part64, part_deg, r1, b1, W2l, W2r):
    def body(p_ref, d_ref, r1_ref, b1_ref, wl_ref, wr_ref, p2_ref, r2_ref):
        agg = p_ref[0] + p_ref[1]
        deg = d_ref[0, :, 0:1] + d_ref[1, :, 0:1]
        invd = 1.0 / jnp.maximum(deg, 1.0)
        h = jnp.maximum(agg * invd + b1_ref[...] + r1_ref[...], 0.0)
        p2_ref[...] = jnp.dot(h, wl_ref[...], preferred_element_type=jnp.float32)
        r2_ref[...] = jnp.dot(h, wr_ref[...], preferred_element_type=jnp.float32)

    return pl.pallas_call(
        body,
        grid=(_NBLK,),
        in_specs=[
            pl.BlockSpec((NC, _RB, 64), lambda i: (0, i, 0)),
            pl.BlockSpec((NC, _RB, 16), lambda i: (0, i, 0)),
            pl.BlockSpec((_RB, 64), lambda i: (i, 0)),
            pl.BlockSpec((1, 64), lambda i: (0, 0)),
            pl.BlockSpec((64, 32), lambda i: (0, 0)),
            pl.BlockSpec((64, 32), lambda i: (0, 0)),
        ],
        out_specs=[
            pl.BlockSpec((_RB, 32), lambda i: (i, 0)),
            pl.BlockSpec((_RB, 32), lambda i: (i, 0)),
        ],
        out_shape=[
            jax.ShapeDtypeStruct((N_NODES, 32), jnp.float32),
            jax.ShapeDtypeStruct((N_NODES, 32), jnp.float32),
        ],
    )(part64, part_deg, r1, b1, W2l, W2r)


def _tc_decoder_tables(part32, part_deg, r2, b2, Wa_s, Wa_d, ba):
    def body(p_ref, d_ref, r2_ref, b2_ref, ws_ref, wd_ref, ba_ref,
             za_ref, zb_ref):
        agg = p_ref[0] + p_ref[1]
        deg = d_ref[0, :, 0:1] + d_ref[1, :, 0:1]
        invd = 1.0 / jnp.maximum(deg, 1.0)
        z = agg * invd + b2_ref[...] + r2_ref[...]
        za_ref[...] = (jnp.dot(z, ws_ref[...], preferred_element_type=jnp.float32)
                       + ba_ref[...])
        zb_ref[...] = jnp.dot(z, wd_ref[...], preferred_element_type=jnp.float32)

    return pl.pallas_call(
        body,
        grid=(_NBLK,),
        in_specs=[
            pl.BlockSpec((NC, _RB, 32), lambda i: (0, i, 0)),
            pl.BlockSpec((NC, _RB, 16), lambda i: (0, i, 0)),
            pl.BlockSpec((_RB, 32), lambda i: (i, 0)),
            pl.BlockSpec((1, 32), lambda i: (0, 0)),
            pl.BlockSpec((32, 32), lambda i: (0, 0)),
            pl.BlockSpec((32, 32), lambda i: (0, 0)),
            pl.BlockSpec((1, 32), lambda i: (0, 0)),
        ],
        out_specs=[
            pl.BlockSpec((_RB, 32), lambda i: (i, 0)),
            pl.BlockSpec((_RB, 32), lambda i: (i, 0)),
        ],
        out_shape=[
            jax.ShapeDtypeStruct((N_NODES, 32), jnp.float32),
            jax.ShapeDtypeStruct((N_NODES, 32), jnp.float32),
        ],
    )(part32, part_deg, r2, b2, Wa_s, Wa_d, ba)


_RB4 = 1024


def _tc_mlp(h1p, Wb4, bb4, Wc4, bc4):
    nrows = h1p.shape[0]
    assert nrows % _RB4 == 0
    rb = 2 * _RB4 if nrows % (2 * _RB4) == 0 else _RB4

    def body(h_ref, wb_ref, bb_ref, wc_ref, bc_ref, f_ref, a_ref, o_ref):
        h2 = jnp.maximum(
            jnp.dot(h_ref[...], wb_ref[...], preferred_element_type=jnp.float32)
            + bb_ref[...], 0.0)
        g = (jnp.dot(h2, wc_ref[...], preferred_element_type=jnp.float32)
             + bc_ref[...])
        t = jnp.dot(g, f_ref[...], preferred_element_type=jnp.float32)
        lane = lax.broadcasted_iota(jnp.int32, (rb, 128), 1)
        row = lax.broadcasted_iota(jnp.int32, (rb, 128), 0)
        sel = (lane // 4 == row % 32).astype(jnp.float32)
        o_ref[...] = jnp.dot(a_ref[...], t * sel,
                             preferred_element_type=jnp.float32)

    lane4 = jnp.arange(128, dtype=jnp.int32) % 4
    Fsp = (lane4[None, :] == jnp.arange(4, dtype=jnp.int32)[:, None]
           ).astype(jnp.float32)
    Asum = (jnp.arange(rb, dtype=jnp.int32)[None, :] // 32
            == jnp.arange(rb // 32, dtype=jnp.int32)[:, None]
            ).astype(jnp.float32)
    return pl.pallas_call(
        body,
        grid=(nrows // rb,),
        in_specs=[
            pl.BlockSpec((rb, 128), lambda i: (i, 0)),
            pl.BlockSpec((128, 64), lambda i: (0, 0)),
            pl.BlockSpec((1, 64), lambda i: (0, 0)),
            pl.BlockSpec((64, 4), lambda i: (0, 0)),
            pl.BlockSpec((1, 4), lambda i: (0, 0)),
            pl.BlockSpec((4, 128), lambda i: (0, 0)),
            pl.BlockSpec((rb // 32, rb), lambda i: (0, 0)),
        ],
        out_specs=pl.BlockSpec((rb // 32, 128), lambda i: (i, 0)),
        out_shape=jax.ShapeDtypeStruct((nrows // 32, 128), jnp.float32),
    )(h1p, Wb4, bb4, Wc4, bc4, Fsp, Asum)


def kernel(x, edge_index, edge_label_index, W1l, b1, W1r, W2l, b2, W2r,
           Wa, ba, Wb, bb, Wc, bc):
    src = _pad_idx(edge_index[0], 0, N_NODES)
    dst = _pad_idx(edge_index[1], N_NODES, NPAD - N_NODES)
    ls = _pad_idx(edge_label_index[0], 0, N_NODES)
    ld = _pad_idx(edge_label_index[1], 0, N_NODES)

    b1r = b1.reshape(1, 64)
    b2r = b2.reshape(1, 32)
    bar = ba.reshape(1, 32)
    z128 = jnp.zeros((32, 16), jnp.float32)
    z64 = jnp.zeros((16, 1), jnp.float32)
    Wb4 = jnp.block([[Wb if i == j else z128 for j in range(4)]
                     for i in range(4)])
    Wc4 = jnp.block([[Wc if i == j else z64 for j in range(4)]
                     for i in range(4)])
    bb4 = jnp.tile(bb, 4).reshape(1, 64)
    bc4 = jnp.tile(bc, 4).reshape(1, 4)

    p1, r1 = _tc_premul(x, W1l, W1r)
    part64, part_deg = _sc_aggregate(p1, src, dst, 64, with_deg=True)
    p2, r2 = _tc_layer2_premul(part64, part_deg, r1, b1r, W2l, W2r)
    (part32,) = _sc_aggregate(p2, src, dst, 32, with_deg=False)
    za, zb = _tc_decoder_tables(part32, part_deg, r2, b2r,
                                Wa[:32], Wa[32:], bar)
    outs = []
    k0 = 0
    for n in (20, 20, 20, 19):
        h1s = _sc_decoder_gather(za, zb, ls, ld, k0, n)
        outs.append(_tc_mlp(h1s, Wb4, bb4, Wc4, bc4))
        k0 += NW * n
    o = jnp.concatenate(outs, axis=0)
    return o[:N_PRED // 128].reshape(N_PRED)
